# Initial kernel scaffold; baseline (speedup 1.0000x reference)
import jax
import jax.numpy as jnp
from jax.experimental import pallas as pl


def kernel(eb_input, eb_offset, W0, W1, W2, W3):
    raise NotImplementedError("write your pallas kernel here")


# speedup vs baseline: 9.2505x; 9.2505x over previous
import functools

import jax
import jax.numpy as jnp
from jax import lax
from jax.experimental import pallas as pl
from jax.experimental.pallas import tpu as pltpu
from jax.experimental.pallas import tpu_sc as plsc

_N_IDX = 819200
_DIM = 3
_NC = 2
_NS = 16
_NW = _NC * _NS
_PER_W = _N_IDX // _NW
_G = 128
_GROUPS = _PER_W // _G

_mesh = plsc.VectorSubcoreMesh(core_axis_name="c", subcore_axis_name="s")


@functools.partial(
    pl.kernel,
    out_type=jax.ShapeDtypeStruct((_NW, 16), jnp.float32),
    mesh=_mesh,
    compiler_params=pltpu.CompilerParams(
        needs_layout_passes=False, use_tc_tiling_on_sc=False),
    scratch_types=[
        pltpu.VMEM((_GROUPS, _G), jnp.int32),
        pltpu.VMEM((_G, 8), jnp.float32),
        pltpu.VMEM((_G, 8), jnp.float32),
        pltpu.VMEM((_G, 8), jnp.float32),
        pltpu.VMEM((_G, 8), jnp.float32),
        pltpu.VMEM((_G, 8), jnp.float32),
        pltpu.VMEM((_G, 8), jnp.float32),
        pltpu.VMEM((_G, 8), jnp.float32),
        pltpu.VMEM((_G, 8), jnp.float32),
        pltpu.VMEM((16,), jnp.float32),
        pltpu.SemaphoreType.DMA,
        pltpu.SemaphoreType.DMA,
    ],
)
def _gather_sum(idx_hbm, w0, w1, w2, w3, out_hbm,
                idx_v, a0, a1, a2, a3, b0, b1, b2, b3, obuf, sem_a, sem_b):
    tables = (w0, w1, w2, w3)
    bufs_a = (a0, a1, a2, a3)
    bufs_b = (b0, b1, b2, b3)
    wid = lax.axis_index("s") * _NC + lax.axis_index("c")
    _IOTA = lax.iota(jnp.int32, 16)

    pltpu.sync_copy(idx_hbm.at[pl.ds(wid * _GROUPS, _GROUPS)], idx_v)

    zeros = jnp.zeros((16,), jnp.float32)

    def fire(g, bufs, sem):
        row = idx_v.at[g]
        return [pltpu.async_copy(t.at[row], b, sem)
                for t, b in zip(tables, bufs)]

    def accum(bufs, accs):
        accs = list(accs)
        for ti, buf in enumerate(bufs):
            for r in range(_G // 16):
                rows = _IOTA + r * 16
                for j in range(_DIM):
                    col = plsc.load_gather(
                        buf, [rows, jnp.full((16,), j, jnp.int32)])
                    accs[ti * _DIM + j] = accs[ti * _DIM + j] + col
        return tuple(accs)

    def body(o, accs):
        cps_a = fire(2 * o, bufs_a, sem_a)
        cps_b = fire(2 * o + 1, bufs_b, sem_b)
        for cp in cps_a:
            cp.wait()
        accs = accum(bufs_a, accs)
        for cp in cps_b:
            cp.wait()
        accs = accum(bufs_b, accs)
        return accs

    accs = lax.fori_loop(0, _GROUPS // 2, body, (zeros,) * 12)

    vals = zeros
    for j in range(_DIM):
        s02 = jnp.sum(accs[0 * _DIM + j] + accs[2 * _DIM + j])
        s13 = jnp.sum(accs[1 * _DIM + j] + accs[3 * _DIM + j])
        vals = jnp.where(_IOTA == j, s02, vals)
        vals = jnp.where(_IOTA == _DIM + j, s13, vals)
    obuf[...] = vals
    pltpu.sync_copy(obuf, out_hbm.at[wid])


def kernel(eb_input, eb_offset, W0, W1, W2, W3):
    del eb_offset
    idx = eb_input.astype(jnp.int32).reshape(_NW * _GROUPS, _G)
    pad = lambda w: jnp.pad(w, ((0, 0), (0, 8 - _DIM)))
    partials = _gather_sum(idx, pad(W0), pad(W1), pad(W2), pad(W3))
    return jnp.sum(partials, axis=0)[:6]


# speedup vs baseline: 450.4561x; 48.6953x over previous
import functools

import jax
import jax.numpy as jnp
from jax import lax
from jax.experimental import pallas as pl
from jax.experimental.pallas import tpu as pltpu
from jax.experimental.pallas import tpu_sc as plsc

_N_IDX = 819200
_DIM = 3
_NC = 2
_NS = 16
_NW = _NC * _NS
_PER_W = _N_IDX // _NW
_G = 128
_GROUPS = _PER_W // _G
_V = 1 << 20
_VOCAB = 1000000
_CHUNK = 2048
_PER_W_V = _V // _NW
_NCHUNK = _PER_W_V // _CHUNK
_SC_SLICE = _V // _NS
_ZB = 16384

_mesh = plsc.VectorSubcoreMesh(core_axis_name="c", subcore_axis_name="s")
_params = pltpu.CompilerParams(
    needs_layout_passes=False, use_tc_tiling_on_sc=False)


@functools.partial(
    pl.kernel,
    out_type=jax.ShapeDtypeStruct((_NC, _V), jnp.float32),
    mesh=_mesh,
    compiler_params=_params,
    scratch_types=[
        pltpu.VMEM((_GROUPS, _G), jnp.int32),
        pltpu.VMEM((_G,), jnp.float32),
        pltpu.VMEM((_ZB,), jnp.float32),
        pltpu.VMEM_SHARED((_V,), jnp.float32),
    ],
)
def _hist(idx_hbm, out_hbm, idx_v, ones_v, zbuf, counts_sp):
    cid = lax.axis_index("c")
    sid = lax.axis_index("s")
    wid = sid * _NC + cid
    one = jnp.full((16,), 1.0, jnp.float32)
    zero = jnp.zeros((16,), jnp.float32)

    pltpu.sync_copy(idx_hbm.at[pl.ds(wid * _GROUPS, _GROUPS)], idx_v)
    for k in range(_G // 16):
        ones_v[pl.ds(k * 16, 16)] = one

    def zfill(i, _):
        zbuf[pl.ds(i * 16, 16)] = zero
        return 0
    lax.fori_loop(0, _ZB // 16, zfill, 0)

    def zcopy(k, _):
        pltpu.sync_copy(
            zbuf, counts_sp.at[pl.ds(sid * _SC_SLICE + k * _ZB, _ZB)])
        return 0
    lax.fori_loop(0, _SC_SLICE // _ZB, zcopy, 0)
    plsc.subcore_barrier()

    def scat(g, _):
        pltpu.sync_copy(ones_v, counts_sp.at[idx_v.at[g]], add=True)
        return 0
    lax.fori_loop(0, _GROUPS, scat, 0)
    plsc.subcore_barrier()

    pltpu.sync_copy(counts_sp.at[pl.ds(sid * _SC_SLICE, _SC_SLICE)],
                    out_hbm.at[cid, pl.ds(sid * _SC_SLICE, _SC_SLICE)])


_wsum_scratch = (
    [pltpu.VMEM((_CHUNK,), jnp.float32) for _ in range(2 * 14)]
    + [pltpu.VMEM((16,), jnp.float32),
       pltpu.SemaphoreType.DMA,
       pltpu.SemaphoreType.DMA]
)


@functools.partial(
    pl.kernel,
    out_type=jax.ShapeDtypeStruct((_NW, 16), jnp.float32),
    mesh=_mesh,
    compiler_params=_params,
    scratch_types=_wsum_scratch,
)
def _wsum(counts_hbm, *rest):
    cols_hbm = rest[:12]
    out_hbm = rest[12]
    bufs_a = rest[13:13 + 14]
    bufs_b = rest[27:27 + 14]
    obuf = rest[41]
    sem_a = rest[42]
    sem_b = rest[43]
    wid = lax.axis_index("s") * _NC + lax.axis_index("c")
    base = wid * _PER_W_V
    iota = lax.iota(jnp.int32, 16)
    zeros = jnp.zeros((16,), jnp.float32)

    def fire(c, bufs, sem):
        off = base + c * _CHUNK
        cps = [pltpu.async_copy(counts_hbm.at[0, pl.ds(off, _CHUNK)],
                                bufs[0], sem),
               pltpu.async_copy(counts_hbm.at[1, pl.ds(off, _CHUNK)],
                                bufs[1], sem)]
        for i, col in enumerate(cols_hbm):
            cps.append(pltpu.async_copy(col.at[pl.ds(off, _CHUNK)],
                                        bufs[2 + i], sem))
        return cps

    def accum(bufs, accs):
        def vec(i, accs):
            sl = pl.ds(i * 16, 16)
            cnt = bufs[0][sl] + bufs[1][sl]
            return tuple(accs[t] + cnt * bufs[2 + t][sl] for t in range(12))
        return lax.fori_loop(0, _CHUNK // 16, vec, accs)

    def body(o, accs):
        cps_a = fire(2 * o, bufs_a, sem_a)
        cps_b = fire(2 * o + 1, bufs_b, sem_b)
        for cp in cps_a:
            cp.wait()
        accs = accum(bufs_a, accs)
        for cp in cps_b:
            cp.wait()
        accs = accum(bufs_b, accs)
        return accs

    accs = lax.fori_loop(0, _NCHUNK // 2, body, (zeros,) * 12)

    vals = zeros
    for j in range(_DIM):
        s02 = jnp.sum(accs[0 * _DIM + j] + accs[2 * _DIM + j])
        s13 = jnp.sum(accs[1 * _DIM + j] + accs[3 * _DIM + j])
        vals = jnp.where(iota == j, s02, vals)
        vals = jnp.where(iota == _DIM + j, s13, vals)
    obuf[...] = vals
    pltpu.sync_copy(obuf, out_hbm.at[wid])


def kernel(eb_input, eb_offset, W0, W1, W2, W3):
    del eb_offset
    idx = eb_input.astype(jnp.int32).reshape(_NW * _GROUPS, _G)
    counts = _hist(idx)
    cols = [jnp.pad(W[:, j], (0, _V - _VOCAB))
            for W in (W0, W1, W2, W3) for j in range(_DIM)]
    partials = _wsum(counts, *cols)
    return jnp.sum(partials, axis=0)[:6]


# speedup vs baseline: 744.8886x; 1.6536x over previous
import functools

import jax
import jax.numpy as jnp
from jax import lax
from jax.experimental import pallas as pl
from jax.experimental.pallas import tpu as pltpu
from jax.experimental.pallas import tpu_sc as plsc

_N_IDX = 819200
_DIM = 3
_NC = 2
_NS = 16
_NW = _NC * _NS
_PER_W = _N_IDX // _NW
_G = 128
_GROUPS = _PER_W // _G
_V = 1 << 20
_VOCAB = 1000000
_CHUNK = 2048
_PER_W_V = _V // _NW
_NCHUNK = _PER_W_V // _CHUNK
_SC_SLICE = _V // _NS
_ZB = 16384

_mesh = plsc.VectorSubcoreMesh(core_axis_name="c", subcore_axis_name="s")
_params = pltpu.CompilerParams(
    needs_layout_passes=False, use_tc_tiling_on_sc=False)


@functools.partial(
    pl.kernel,
    out_type=jax.ShapeDtypeStruct((_NC, _V), jnp.float32),
    mesh=_mesh,
    compiler_params=_params,
    scratch_types=[
        pltpu.VMEM((_GROUPS, _G), jnp.int32),
        pltpu.VMEM((_G,), jnp.float32),
        pltpu.VMEM((_ZB,), jnp.float32),
        pltpu.VMEM_SHARED((_V,), jnp.float32),
    ],
)
def _hist(idx_hbm, out_hbm, idx_v, ones_v, zbuf, counts_sp):
    cid = lax.axis_index("c")
    sid = lax.axis_index("s")
    wid = sid * _NC + cid
    one = jnp.full((16,), 1.0, jnp.float32)
    zero = jnp.zeros((16,), jnp.float32)

    pltpu.sync_copy(idx_hbm.at[pl.ds(wid * _GROUPS, _GROUPS)], idx_v)
    for k in range(_G // 16):
        ones_v[pl.ds(k * 16, 16)] = one

    def zfill(i, _):
        zbuf[pl.ds(i * 16, 16)] = zero
        return 0
    lax.fori_loop(0, _ZB // 16, zfill, 0)

    def zcopy(k, _):
        pltpu.sync_copy(
            zbuf, counts_sp.at[pl.ds(sid * _SC_SLICE + k * _ZB, _ZB)])
        return 0
    lax.fori_loop(0, _SC_SLICE // _ZB, zcopy, 0)
    plsc.subcore_barrier()

    def scat(g, _):
        pltpu.sync_copy(ones_v, counts_sp.at[idx_v.at[g]], add=True)
        return 0
    lax.fori_loop(0, _GROUPS, scat, 0)
    plsc.subcore_barrier()

    pltpu.sync_copy(counts_sp.at[pl.ds(sid * _SC_SLICE, _SC_SLICE)],
                    out_hbm.at[cid, pl.ds(sid * _SC_SLICE, _SC_SLICE)])


_wsum_scratch = (
    [pltpu.VMEM((_CHUNK,), jnp.float32) for _ in range(2 * 8)]
    + [pltpu.VMEM((16,), jnp.float32),
       pltpu.SemaphoreType.DMA,
       pltpu.SemaphoreType.DMA]
)


@functools.partial(
    pl.kernel,
    out_type=jax.ShapeDtypeStruct((_NW, 16), jnp.float32),
    mesh=_mesh,
    compiler_params=_params,
    scratch_types=_wsum_scratch,
)
def _wsum(counts_hbm, *rest):
    cols_hbm = rest[:6]
    out_hbm = rest[6]
    bufs_a = rest[7:7 + 8]
    bufs_b = rest[15:15 + 8]
    obuf = rest[23]
    sem_a = rest[24]
    sem_b = rest[25]
    wid = lax.axis_index("s") * _NC + lax.axis_index("c")
    base = wid * _PER_W_V
    iota = lax.iota(jnp.int32, 16)
    zeros = jnp.zeros((16,), jnp.float32)

    def fire(c, bufs, sem):
        off = base + c * _CHUNK
        cps = [pltpu.async_copy(counts_hbm.at[0, pl.ds(off, _CHUNK)],
                                bufs[0], sem),
               pltpu.async_copy(counts_hbm.at[1, pl.ds(off, _CHUNK)],
                                bufs[1], sem)]
        for i, col in enumerate(cols_hbm):
            cps.append(pltpu.async_copy(col.at[pl.ds(off, _CHUNK)],
                                        bufs[2 + i], sem))
        return cps

    def accum(bufs, accs):
        def vec(i, accs):
            sl = pl.ds(i * 16, 16)
            cnt = bufs[0][sl] + bufs[1][sl]
            return tuple(accs[t] + cnt * bufs[2 + t][sl] for t in range(6))
        return lax.fori_loop(0, _CHUNK // 16, vec, accs)

    def body(o, accs):
        cps_a = fire(2 * o, bufs_a, sem_a)
        cps_b = fire(2 * o + 1, bufs_b, sem_b)
        for cp in cps_a:
            cp.wait()
        accs = accum(bufs_a, accs)
        for cp in cps_b:
            cp.wait()
        accs = accum(bufs_b, accs)
        return accs

    accs = lax.fori_loop(0, _NCHUNK // 2, body, (zeros,) * 6)

    vals = zeros
    for t in range(6):
        vals = jnp.where(iota == t, jnp.sum(accs[t]), vals)
    obuf[...] = vals
    pltpu.sync_copy(obuf, out_hbm.at[wid])


def kernel(eb_input, eb_offset, W0, W1, W2, W3):
    del eb_offset
    idx = eb_input.astype(jnp.int32).reshape(_NW * _GROUPS, _G)
    counts = _hist(idx)
    cols = [jnp.pad(Wa[:, j] + Wb[:, j], (0, _V - _VOCAB))
            for (Wa, Wb) in ((W0, W2), (W1, W3)) for j in range(_DIM)]
    partials = _wsum(counts, *cols)
    return jnp.sum(partials, axis=0)[:6]


# speedup vs baseline: 744.9696x; 1.0001x over previous
import functools

import jax
import jax.numpy as jnp
from jax import lax
from jax.experimental import pallas as pl
from jax.experimental.pallas import tpu as pltpu
from jax.experimental.pallas import tpu_sc as plsc

_N_IDX = 819200
_DIM = 3
_NC = 2
_NS = 16
_NW = _NC * _NS
_PER_W = _N_IDX // _NW
_G = 512
_GROUPS = _PER_W // _G
_V = 1 << 20
_VOCAB = 1000000
_CHUNK = 2048
_PER_W_V = _V // _NW
_NCHUNK = _PER_W_V // _CHUNK
_SC_SLICE = _V // _NS
_ZB = 16384

_mesh = plsc.VectorSubcoreMesh(core_axis_name="c", subcore_axis_name="s")
_params = pltpu.CompilerParams(
    needs_layout_passes=False, use_tc_tiling_on_sc=False)


@functools.partial(
    pl.kernel,
    out_type=jax.ShapeDtypeStruct((_NC, _V), jnp.float32),
    mesh=_mesh,
    compiler_params=_params,
    scratch_types=[
        pltpu.VMEM((_GROUPS, _G), jnp.int32),
        pltpu.VMEM((_G,), jnp.float32),
        pltpu.VMEM((_ZB,), jnp.float32),
        pltpu.VMEM_SHARED((_V,), jnp.float32),
    ],
)
def _hist(idx_hbm, out_hbm, idx_v, ones_v, zbuf, counts_sp):
    cid = lax.axis_index("c")
    sid = lax.axis_index("s")
    wid = sid * _NC + cid
    one = jnp.full((16,), 1.0, jnp.float32)
    zero = jnp.zeros((16,), jnp.float32)

    pltpu.sync_copy(idx_hbm.at[pl.ds(wid * _GROUPS, _GROUPS)], idx_v)
    for k in range(_G // 16):
        ones_v[pl.ds(k * 16, 16)] = one

    def zfill(i, _):
        zbuf[pl.ds(i * 16, 16)] = zero
        return 0
    lax.fori_loop(0, _ZB // 16, zfill, 0)

    def zcopy(k, _):
        pltpu.sync_copy(
            zbuf, counts_sp.at[pl.ds(sid * _SC_SLICE + k * _ZB, _ZB)])
        return 0
    lax.fori_loop(0, _SC_SLICE // _ZB, zcopy, 0)
    plsc.subcore_barrier()

    def scat(g, _):
        pltpu.sync_copy(ones_v, counts_sp.at[idx_v.at[g]], add=True)
        return 0
    lax.fori_loop(0, _GROUPS, scat, 0)
    plsc.subcore_barrier()

    pltpu.sync_copy(counts_sp.at[pl.ds(sid * _SC_SLICE, _SC_SLICE)],
                    out_hbm.at[cid, pl.ds(sid * _SC_SLICE, _SC_SLICE)])


_wsum_scratch = (
    [pltpu.VMEM((_CHUNK,), jnp.float32) for _ in range(2 * 8)]
    + [pltpu.VMEM((16,), jnp.float32),
       pltpu.SemaphoreType.DMA,
       pltpu.SemaphoreType.DMA]
)


@functools.partial(
    pl.kernel,
    out_type=jax.ShapeDtypeStruct((_NW, 16), jnp.float32),
    mesh=_mesh,
    compiler_params=_params,
    scratch_types=_wsum_scratch,
)
def _wsum(counts_hbm, *rest):
    cols_hbm = rest[:6]
    out_hbm = rest[6]
    bufs_a = rest[7:7 + 8]
    bufs_b = rest[15:15 + 8]
    obuf = rest[23]
    sem_a = rest[24]
    sem_b = rest[25]
    wid = lax.axis_index("s") * _NC + lax.axis_index("c")
    base = wid * _PER_W_V
    iota = lax.iota(jnp.int32, 16)
    zeros = jnp.zeros((16,), jnp.float32)

    def fire(c, bufs, sem):
        off = base + c * _CHUNK
        cps = [pltpu.async_copy(counts_hbm.at[0, pl.ds(off, _CHUNK)],
                                bufs[0], sem),
               pltpu.async_copy(counts_hbm.at[1, pl.ds(off, _CHUNK)],
                                bufs[1], sem)]
        for i, col in enumerate(cols_hbm):
            cps.append(pltpu.async_copy(col.at[pl.ds(off, _CHUNK)],
                                        bufs[2 + i], sem))
        return cps

    def accum(bufs, accs):
        def vec(i, accs):
            sl = pl.ds(i * 16, 16)
            cnt = bufs[0][sl] + bufs[1][sl]
            return tuple(accs[t] + cnt * bufs[2 + t][sl] for t in range(6))
        return lax.fori_loop(0, _CHUNK // 16, vec, accs)

    def body(o, accs):
        cps_a = fire(2 * o, bufs_a, sem_a)
        cps_b = fire(2 * o + 1, bufs_b, sem_b)
        for cp in cps_a:
            cp.wait()
        accs = accum(bufs_a, accs)
        for cp in cps_b:
            cp.wait()
        accs = accum(bufs_b, accs)
        return accs

    accs = lax.fori_loop(0, _NCHUNK // 2, body, (zeros,) * 6)

    vals = zeros
    for t in range(6):
        vals = jnp.where(iota == t, jnp.sum(accs[t]), vals)
    obuf[...] = vals
    pltpu.sync_copy(obuf, out_hbm.at[wid])


def kernel(eb_input, eb_offset, W0, W1, W2, W3):
    del eb_offset
    idx = eb_input.astype(jnp.int32).reshape(_NW * _GROUPS, _G)
    counts = _hist(idx)
    cols = [jnp.pad(Wa[:, j] + Wb[:, j], (0, _V - _VOCAB))
            for (Wa, Wb) in ((W0, W2), (W1, W3)) for j in range(_DIM)]
    partials = _wsum(counts, *cols)
    return jnp.sum(partials, axis=0)[:6]
